# baseline (speedup 1.0000x reference)
import math

import jax
import jax.numpy as jnp
from jax.experimental import pallas as pl

_TAU = 0.2
_K = 15
_LOG2E = math.log2(math.e)


def _normalize_rows(x, scale=1.0):
    ss = jnp.sum(x * x, axis=-1, keepdims=True)
    n = jnp.maximum(jnp.sqrt(ss), 1e-12)
    return x * (scale / n)


def _normalize_y_kernel(x_ref, o_ref):
    o_ref[...] = _normalize_rows(x_ref[...])


def _simtopk_kernel(x_ref, yn_ref, o_ref):
    x = _normalize_rows(x_ref[0])
    y = yn_ref[0]
    s = jax.lax.dot_general(
        x, y, (((1,), (1,)), ((), ())),
        preferred_element_type=jnp.float32,
    ) / _TAU
    m = jnp.max(s, axis=-1, keepdims=True)
    for _ in range(_K - 1):
        m = jnp.max(jnp.where(s < m, s, -jnp.inf), axis=-1, keepdims=True)
    e = jnp.where(s >= m, jnp.exp2(s * _LOG2E), 0.0)
    o_ref[0] = e / jnp.sum(e, axis=-1, keepdims=True)


def kernel(feat_x, feat_y):
    B, Nx, C = feat_x.shape
    Ny = feat_y.shape[1]
    BX = 256

    yn = pl.pallas_call(
        _normalize_y_kernel,
        grid=(B,),
        in_specs=[pl.BlockSpec((1, Ny, C), lambda b: (b, 0, 0))],
        out_specs=pl.BlockSpec((1, Ny, C), lambda b: (b, 0, 0)),
        out_shape=jax.ShapeDtypeStruct((B, Ny, C), jnp.float32),
    )(feat_y)

    out = pl.pallas_call(
        _simtopk_kernel,
        grid=(B, Nx // BX),
        in_specs=[
            pl.BlockSpec((1, BX, C), lambda b, i: (b, i, 0)),
            pl.BlockSpec((1, Ny, C), lambda b, i: (b, 0, 0)),
        ],
        out_specs=pl.BlockSpec((1, BX, Ny), lambda b, i: (b, i, 0)),
        out_shape=jax.ShapeDtypeStruct((B, Nx, Ny), jnp.float32),
    )(feat_x, yn)
    return out


# speedup vs baseline: 1.0093x; 1.0093x over previous
import math

import jax
import jax.numpy as jnp
from jax.experimental import pallas as pl

_TAU = 0.2
_K = 15
_LOG2E = math.log2(math.e)


def _normalize_rows(x):
    ss = jnp.sum(x * x, axis=-1, keepdims=True)
    n = jnp.maximum(jnp.sqrt(ss), 1e-12)
    return x / n


def _normalize_y_kernel(x_ref, o_ref):
    o_ref[...] = _normalize_rows(x_ref[...])


def _simtopk_kernel(x_ref, yn_ref, o_ref):
    x = _normalize_rows(x_ref[0])
    y = yn_ref[0]
    s = jax.lax.dot_general(
        x, y, (((1,), (1,)), ((), ())),
        preferred_element_type=jnp.float32,
    ) / _TAU
    m = jnp.max(s, axis=-1, keepdims=True)
    tops = [m]
    for _ in range(_K - 1):
        m = jnp.max(jnp.where(s < m, s, -jnp.inf), axis=-1, keepdims=True)
        tops.append(m)
    exps = [jnp.exp2(mi * _LOG2E) for mi in tops]
    while len(exps) > 1:
        exps = [a + b for a, b in zip(exps[::2], exps[1::2])] + (
            [exps[-1]] if len(exps) % 2 else [])
    d2 = jnp.log2(exps[0])
    o_ref[0] = jnp.where(s >= m, jnp.exp2(s * _LOG2E - d2), 0.0)


def kernel(feat_x, feat_y):
    B, Nx, C = feat_x.shape
    Ny = feat_y.shape[1]
    BX = 512

    yn = pl.pallas_call(
        _normalize_y_kernel,
        grid=(B,),
        in_specs=[pl.BlockSpec((1, Ny, C), lambda b: (b, 0, 0))],
        out_specs=pl.BlockSpec((1, Ny, C), lambda b: (b, 0, 0)),
        out_shape=jax.ShapeDtypeStruct((B, Ny, C), jnp.float32),
    )(feat_y)

    out = pl.pallas_call(
        _simtopk_kernel,
        grid=(B, Nx // BX),
        in_specs=[
            pl.BlockSpec((1, BX, C), lambda b, i: (b, i, 0)),
            pl.BlockSpec((1, Ny, C), lambda b, i: (b, 0, 0)),
        ],
        out_specs=pl.BlockSpec((1, BX, Ny), lambda b, i: (b, i, 0)),
        out_shape=jax.ShapeDtypeStruct((B, Nx, Ny), jnp.float32),
    )(feat_x, yn)
    return out


# speedup vs baseline: 1.0848x; 1.0748x over previous
import math

import jax
import jax.numpy as jnp
from jax.experimental import pallas as pl
from jax.experimental.pallas import tpu as pltpu

_TAU = 0.2
_K = 15
_LOG2E = math.log2(math.e)


def _normalize_rows(x):
    ss = jnp.sum(x * x, axis=-1, keepdims=True)
    n = jnp.maximum(jnp.sqrt(ss), 1e-12)
    return x / n


def _simtopk_kernel(x_ref, y_ref, o_ref, yn_ref):
    @pl.when(pl.program_id(1) == 0)
    def _():
        yn_ref[...] = _normalize_rows(y_ref[0])

    x = _normalize_rows(x_ref[0])
    s = jax.lax.dot_general(
        x, yn_ref[...], (((1,), (1,)), ((), ())),
        preferred_element_type=jnp.float32,
    ) / _TAU
    m = jnp.max(s, axis=-1, keepdims=True)
    tops = [m]
    for _ in range(_K - 1):
        m = jnp.max(jnp.where(s < m, s, -jnp.inf), axis=-1, keepdims=True)
        tops.append(m)
    exps = [jnp.exp2(mi * _LOG2E) for mi in tops]
    while len(exps) > 1:
        exps = [a + b for a, b in zip(exps[::2], exps[1::2])] + (
            [exps[-1]] if len(exps) % 2 else [])
    d2 = jnp.log2(exps[0])
    o_ref[0] = jnp.where(s >= m, jnp.exp2(s * _LOG2E - d2), 0.0)


def kernel(feat_x, feat_y):
    B, Nx, C = feat_x.shape
    Ny = feat_y.shape[1]
    BX = 512

    out = pl.pallas_call(
        _simtopk_kernel,
        grid=(B, Nx // BX),
        in_specs=[
            pl.BlockSpec((1, BX, C), lambda b, i: (b, i, 0)),
            pl.BlockSpec((1, Ny, C), lambda b, i: (b, 0, 0)),
        ],
        out_specs=pl.BlockSpec((1, BX, Ny), lambda b, i: (b, i, 0)),
        out_shape=jax.ShapeDtypeStruct((B, Nx, Ny), jnp.float32),
        scratch_shapes=[pltpu.VMEM((Ny, C), jnp.float32)],
        compiler_params=pltpu.CompilerParams(
            dimension_semantics=("arbitrary", "arbitrary"),
        ),
    )(feat_x, feat_y)
    return out
